# baseline (speedup 1.0000x reference)
import functools

import jax
import jax.numpy as jnp
from jax import lax
from jax.experimental import pallas as pl
from jax.experimental.pallas import tpu as pltpu
from jax.experimental.pallas import tpu_sc as plsc

NUM_EMB = 1000000
DIM = 32
BATCH = 16384
SEQ = 50
BB = 128

NUM_WORKERS = 32
IBLK = BATCH // NUM_WORKERS
NSUB = IBLK // BB
NBUF = 4


def _make_lookup():
    mesh = plsc.VectorSubcoreMesh(core_axis_name="c", subcore_axis_name="s")

    @functools.partial(
        pl.kernel,
        out_type=jax.ShapeDtypeStruct(
            (SEQ, DIM // 8, BATCH // BB, 8, BB), jnp.float32
        ),
        mesh=mesh,
        scratch_types=[
            pltpu.VMEM((IBLK, SEQ), jnp.int32),
            pltpu.VMEM((SEQ * IBLK,), jnp.int32),
            [pltpu.VMEM((BB, DIM), jnp.float32) for _ in range(NBUF)],
            [pltpu.VMEM((DIM, BB), jnp.float32) for _ in range(NBUF)],
            [pltpu.SemaphoreType.DMA for _ in range(NBUF)],
            [pltpu.SemaphoreType.DMA for _ in range(NBUF)],
        ],
        compiler_params=pltpu.CompilerParams(
            use_tc_tiling_on_sc=False, needs_layout_passes=False
        ),
    )
    def lookup(table_hbm, idx_hbm, out_hbm, idxb, idxT, rowbuf, colbuf, gsem, ssem):
        wid = lax.axis_index("s") * 2 + lax.axis_index("c")
        i0 = wid * IBLK
        pltpu.sync_copy(idx_hbm.at[pl.ds(i0, IBLK), :], idxb)

        lanes = lax.iota(jnp.int32, 16)
        rowiv = [g * 16 + lanes for g in range(BB // 16)]

        @pl.loop(0, SEQ * IBLK // 16)
        def _t(v):
            f = v * 16 + lanes
            j = jnp.right_shift(f, 9)
            i = jnp.bitwise_and(f, IBLK - 1)
            idxT[pl.ds(v * 16, 16)] = plsc.load_gather(idxb, [i, j])

        def start_gather(j, b):
            pltpu.async_copy(
                table_hbm.at[idxT.at[pl.ds(j * IBLK + b * BB, BB)]],
                rowbuf[b],
                gsem[b],
            )

        def extract(b):
            @pl.loop(0, DIM)
            def _d(d):
                for g in range(BB // 16):
                    colbuf[b][d, pl.ds(g * 16, 16)] = plsc.load_gather(
                        rowbuf[b], [rowiv[g], lanes * 0 + d]
                    )

        def start_store(j, b):
            it = wid * NSUB + b
            for dt in range(DIM // 8):
                pltpu.async_copy(
                    colbuf[b].at[pl.ds(dt * 8, 8), :],
                    out_hbm.at[j, dt, it, :, :],
                    ssem[b],
                )

        def wait_gather(b):
            pltpu.make_async_copy(
                table_hbm.at[pl.ds(0, BB), :], rowbuf[b], gsem[b]
            ).wait()

        def wait_store(b):
            for dt in range(DIM // 8):
                pltpu.make_async_copy(
                    out_hbm.at[0, dt, 0, :, :],
                    colbuf[b].at[pl.ds(dt * 8, 8), :],
                    ssem[b],
                ).wait()

        for b in range(NBUF):
            start_gather(0, b)

        @pl.loop(0, SEQ)
        def _ring(g):
            for b in range(NBUF):
                wait_gather(b)

                @pl.when(g >= 1)
                def _():
                    wait_store(b)

                extract(b)
                start_store(g, b)

                @pl.when(g < SEQ - 1)
                def _():
                    start_gather(g + 1, b)

        for b in range(NBUF):
            wait_store(b)

    return lookup


_lookup = _make_lookup()


def kernel(weight, indices):
    out5d = _lookup(weight, indices.astype(jnp.int32))
    return out5d.transpose(2, 4, 0, 1, 3).reshape(BATCH, SEQ, DIM)


# speedup vs baseline: 1.3395x; 1.3395x over previous
import functools

import jax
import jax.numpy as jnp
from jax import lax
from jax.experimental import pallas as pl
from jax.experimental.pallas import tpu as pltpu
from jax.experimental.pallas import tpu_sc as plsc

NUM_EMB = 1000000
DIM = 32
BATCH = 16384
SEQ = 50
BB = 128

NUM_WORKERS = 32
IBLK = BATCH // NUM_WORKERS
NSUB = IBLK // BB
NBUF = 4


def _make_lookup():
    mesh = plsc.VectorSubcoreMesh(core_axis_name="c", subcore_axis_name="s")

    @functools.partial(
        pl.kernel,
        out_type=jax.ShapeDtypeStruct(
            (SEQ, DIM // 8, BATCH // BB, 8, BB), jnp.float32
        ),
        mesh=mesh,
        scratch_types=[
            pltpu.VMEM((IBLK, SEQ), jnp.int32),
            pltpu.VMEM((SEQ * IBLK,), jnp.int32),
            [pltpu.VMEM((BB, DIM), jnp.float32) for _ in range(NBUF)],
            [pltpu.VMEM((DIM, BB), jnp.float32) for _ in range(NBUF)],
            [pltpu.SemaphoreType.DMA for _ in range(NBUF)],
            [pltpu.SemaphoreType.DMA for _ in range(NBUF)],
        ],
        compiler_params=pltpu.CompilerParams(
            use_tc_tiling_on_sc=False, needs_layout_passes=False
        ),
    )
    def lookup(table_hbm, idx_hbm, out_hbm, idxb, idxT, rowbuf, colbuf, gsem, ssem):
        wid = lax.axis_index("s") * 2 + lax.axis_index("c")
        i0 = wid * IBLK
        pltpu.sync_copy(idx_hbm.at[pl.ds(i0, IBLK), :], idxb)

        lanes = lax.iota(jnp.int32, 16)
        rowiv = [g * 16 + lanes for g in range(BB // 16)]

        @pl.loop(0, SEQ * IBLK // 16)
        def _t(v):
            f = v * 16 + lanes
            j = jnp.right_shift(f, 9)
            i = jnp.bitwise_and(f, IBLK - 1)
            idxT[pl.ds(v * 16, 16)] = plsc.load_gather(idxb, [i, j])

        def start_gather(j, b):
            pltpu.async_copy(
                table_hbm.at[idxT.at[pl.ds(j * IBLK + b * BB, BB)]],
                rowbuf[b],
                gsem[b],
            )

        def extract(b):
            @plsc.parallel_loop(0, DIM, unroll=4)
            def _d(d):
                for g in range(BB // 16):
                    colbuf[b][d, pl.ds(g * 16, 16)] = plsc.load_gather(
                        rowbuf[b], [rowiv[g], lanes * 0 + d]
                    )

        def start_store(j, b):
            it = wid * NSUB + b
            for dt in range(DIM // 8):
                pltpu.async_copy(
                    colbuf[b].at[pl.ds(dt * 8, 8), :],
                    out_hbm.at[j, dt, it, :, :],
                    ssem[b],
                )

        def wait_gather(b):
            pltpu.make_async_copy(
                table_hbm.at[pl.ds(0, BB), :], rowbuf[b], gsem[b]
            ).wait()

        def wait_store(b):
            for dt in range(DIM // 8):
                pltpu.make_async_copy(
                    out_hbm.at[0, dt, 0, :, :],
                    colbuf[b].at[pl.ds(dt * 8, 8), :],
                    ssem[b],
                ).wait()

        for b in range(NBUF):
            start_gather(0, b)

        @pl.loop(0, SEQ)
        def _ring(g):
            for b in range(NBUF):
                wait_gather(b)

                @pl.when(g >= 1)
                def _():
                    wait_store(b)

                extract(b)
                start_store(g, b)

                @pl.when(g < SEQ - 1)
                def _():
                    start_gather(g + 1, b)

        for b in range(NBUF):
            wait_store(b)

    return lookup


_lookup = _make_lookup()


def kernel(weight, indices):
    out5d = _lookup(weight, indices.astype(jnp.int32))
    return out5d.transpose(2, 4, 0, 1, 3).reshape(BATCH, SEQ, DIM)
